# baseline (speedup 1.0000x reference)
import functools

import jax
import jax.numpy as jnp
from jax import lax
from jax.experimental import pallas as pl
from jax.experimental.pallas import tpu as pltpu
from jax.experimental.pallas import tpu_sc as plsc

_SDR = 2048
_CONT = 384
_COLS = 2432
_ROWS_OUT = 4672
_TILE = 16
_NT = _ROWS_OUT // _TILE
_NC = 2
_NS = 16
_NW = _NC * _NS
_KMAX = (_NT + _NW - 1) // _NW


def _sc_body(sdrs, conts, zeros, out, sbuf, cbuf, zbuf):
    wid = lax.axis_index("s") * _NC + lax.axis_index("c")
    pltpu.sync_copy(zeros, zbuf)
    for k in range(_KMAX):
        t = wid + _NW * k
        r0 = t * _TILE

        @pl.when(t <= 255)
        def _():
            off = jnp.where(t < 4, 4032,
                            jnp.where(t < 32, 3520,
                                      jnp.where(t < 36, 3008, -576)))
            src = r0 + off
            pltpu.sync_copy(sdrs.at[pl.ds(src, _TILE)], sbuf)
            pltpu.sync_copy(conts.at[pl.ds(src, _TILE)], cbuf)
            pltpu.sync_copy(sbuf, out.at[pl.ds(r0, _TILE), pl.ds(0, _SDR)])
            pltpu.sync_copy(cbuf, out.at[pl.ds(r0, _TILE), pl.ds(_SDR, _CONT)])

        @pl.when(t == 256)
        def _():
            pltpu.sync_copy(zbuf, out.at[pl.ds(4096, _TILE)])
            pltpu.sync_copy(sdrs.at[pl.ds(3520, 1)], sbuf.at[pl.ds(0, 1)])
            pltpu.sync_copy(conts.at[pl.ds(3520, 1)], cbuf.at[pl.ds(0, 1)])
            pltpu.sync_copy(sbuf.at[pl.ds(0, 1)],
                            out.at[pl.ds(4096, 1), pl.ds(0, _SDR)])
            pltpu.sync_copy(cbuf.at[pl.ds(0, 1)],
                            out.at[pl.ds(4096, 1), pl.ds(_SDR, _CONT)])

        @pl.when((t >= 257) & (t < _NT))
        def _():
            pltpu.sync_copy(zbuf, out.at[pl.ds(r0, _TILE)])


def kernel(sdrs, contents):
    zeros = jnp.zeros((_TILE, _COLS), jnp.float32)
    mesh = plsc.VectorSubcoreMesh(core_axis_name="c", subcore_axis_name="s")
    run = functools.partial(
        pl.kernel,
        mesh=mesh,
        out_type=jax.ShapeDtypeStruct((_ROWS_OUT, _COLS), jnp.float32),
        scratch_types=[
            pltpu.VMEM((_TILE, _SDR), jnp.float32),
            pltpu.VMEM((_TILE, _CONT), jnp.float32),
            pltpu.VMEM((_TILE, _COLS), jnp.float32),
        ],
    )(_sc_body)
    return run(sdrs, contents, zeros)
